# baseline (speedup 1.0000x reference)
import jax
import jax.numpy as jnp
from jax.experimental import pallas as pl
from jax.experimental.pallas import tpu as pltpu

_CH = 128
_NBUF = 6
_BM = 2048


def _body(params_ref, x_ref, adj_ref, h_ref, ln_ref, logits_ref,
          ab_s, zb_s, disr_s, zwb_s, buf_s, sem):
    N = adj_ref.shape[0]
    nc = N // _CH
    nb = N // _BM

    mlpw = params_ref[128:256, 0:40]
    w0 = params_ref[256:384, :]
    b0 = params_ref[384:385, :]
    bdec = params_ref[385:386, 0:40]
    mlpb = params_ref[386:387, 0:40]
    la = params_ref[387:388, 0:1]

    def _start(k, slot):
        pltpu.make_async_copy(adj_ref.at[pl.ds(k * _CH, _CH), :],
                              buf_s.at[slot], sem.at[slot]).start()

    for s in range(_NBUF):
        _start(s, s)

    def _p0(k, carry):
        slot = jax.lax.rem(k, _NBUF)
        pltpu.make_async_copy(adj_ref.at[pl.ds(k * _CH, _CH), :],
                              buf_s.at[slot], sem.at[slot]).wait()
        a = buf_s[slot]
        r0 = pl.multiple_of(k * _CH, _CH)
        deg = jnp.sum(a, axis=1, keepdims=True) + la
        dis = jnp.where(deg > 0, jax.lax.rsqrt(deg), 0.0)
        ab_s[pl.ds(r0, _CH), :] = (a * dis).astype(jnp.bfloat16)
        y = jnp.dot(x_ref[pl.ds(r0, _CH), :], w0,
                    preferred_element_type=jnp.float32)
        z = dis * y
        zb_s[pl.ds(r0, _CH), :] = z.astype(jnp.bfloat16)
        disr_s[pl.ds(r0, _CH), :] = jnp.broadcast_to(dis, z.shape)

        @pl.when(k + _NBUF < nc)
        def _():
            _start(k + _NBUF, slot)

        return carry

    jax.lax.fori_loop(0, nc, _p0, 0)

    def _p1(i, carry):
        r0 = pl.multiple_of(i * _BM, _BM)
        ab = ab_s[pl.ds(r0, _BM), :]
        acc = jnp.dot(ab, zb_s[...], preferred_element_type=jnp.float32)
        dis = disr_s[pl.ds(r0, _BM), :]
        zblk = zb_s[pl.ds(r0, _BM), :].astype(jnp.float32)
        emb = jnp.maximum(acc + la * dis * zblk + b0, 0.0)
        h_ref[pl.ds(r0, _BM), :] = emb
        ln_ref[pl.ds(r0, _BM), :] = jnp.dot(
            emb, mlpw, preferred_element_type=jnp.float32) + mlpb
        wdec_p = params_ref[0:128, :]
        w = jnp.dot(emb, wdec_p, preferred_element_type=jnp.float32)
        zw = dis * w
        zwb_s[pl.ds(r0, _BM), :] = zw.astype(jnp.bfloat16)
        return carry

    jax.lax.fori_loop(0, nb, _p1, 0)

    def _p2(i, carry):
        r0 = pl.multiple_of(i * _BM, _BM)
        ab = ab_s[pl.ds(r0, _BM), :]
        acc = jnp.dot(ab, zwb_s[...], preferred_element_type=jnp.float32)
        nc_ = logits_ref.shape[1]
        zw = zwb_s[pl.ds(r0, _BM), 0:nc_].astype(jnp.float32)
        disc = disr_s[pl.ds(r0, _BM), 0:nc_]
        logits_ref[pl.ds(r0, _BM), :] = acc[:, 0:nc_] + la * disc * zw + bdec
        return carry

    jax.lax.fori_loop(0, nb, _p2, 0)


def kernel(x, adj, loop_att, W0, b0, W_dec, b_dec, mlp_W, mlp_b):
    N, D = x.shape[1], x.shape[2]
    C = W_dec.shape[1]

    A = adj.reshape(N, N)
    x2 = x.reshape(N, D)

    P = jnp.zeros((392, 128), jnp.float32)
    P = P.at[0:128, 0:C].set(W_dec)
    P = P.at[128:256, 0:C].set(mlp_W)
    P = P.at[256:384, :].set(W0)
    P = P.at[384, :].set(b0)
    P = P.at[385, 0:C].set(b_dec)
    P = P.at[386, 0:C].set(mlp_b)
    P = P.at[387, 0].set(loop_att)

    h, ln, logits = pl.pallas_call(
        _body,
        in_specs=[
            pl.BlockSpec(memory_space=pltpu.MemorySpace.VMEM),
            pl.BlockSpec(memory_space=pltpu.MemorySpace.VMEM),
            pl.BlockSpec(memory_space=pltpu.MemorySpace.HBM),
        ],
        out_specs=[
            pl.BlockSpec(memory_space=pltpu.MemorySpace.VMEM),
            pl.BlockSpec(memory_space=pltpu.MemorySpace.VMEM),
            pl.BlockSpec(memory_space=pltpu.MemorySpace.VMEM),
        ],
        out_shape=[
            jax.ShapeDtypeStruct((N, D), jnp.float32),
            jax.ShapeDtypeStruct((N, C), jnp.float32),
            jax.ShapeDtypeStruct((N, C), jnp.float32),
        ],
        scratch_shapes=[
            pltpu.VMEM((N, N), jnp.bfloat16),
            pltpu.VMEM((N, D), jnp.bfloat16),
            pltpu.VMEM((N, D), jnp.float32),
            pltpu.VMEM((N, 128), jnp.bfloat16),
            pltpu.VMEM((_NBUF, _CH, N), jnp.float32),
            pltpu.SemaphoreType.DMA((_NBUF,)),
        ],
    )(P, x2, A)

    return (logits, ln[None], h)


# speedup vs baseline: 1.0349x; 1.0349x over previous
import jax
import jax.numpy as jnp
from jax.experimental import pallas as pl
from jax.experimental.pallas import tpu as pltpu

_CH = 256
_NBUF = 3
_BM = 2048


def _body(params_ref, x_ref, adj_ref, h_ref, ln_ref, logits_ref,
          ab_s, zb_s, disr_s, zwb_s, buf_s, sem):
    N = adj_ref.shape[0]
    nc = N // _CH
    nb = N // _BM

    mlpw = params_ref[128:256, 0:40]
    w0 = params_ref[256:384, :]
    b0 = params_ref[384:385, :]
    bdec = params_ref[385:386, 0:40]
    mlpb = params_ref[386:387, 0:40]
    la = params_ref[387:388, 0:1]

    def _start(k, slot):
        pltpu.make_async_copy(adj_ref.at[pl.ds(k * _CH, _CH), :],
                              buf_s.at[slot], sem.at[slot]).start()

    for s in range(_NBUF):
        _start(s, s)

    def _p0(k, carry):
        slot = jax.lax.rem(k, _NBUF)
        pltpu.make_async_copy(adj_ref.at[pl.ds(k * _CH, _CH), :],
                              buf_s.at[slot], sem.at[slot]).wait()
        a = buf_s[slot]
        r0 = pl.multiple_of(k * _CH, _CH)
        deg = jnp.sum(a, axis=1, keepdims=True) + la
        dis = jnp.where(deg > 0, jax.lax.rsqrt(deg), 0.0)
        ab_s[pl.ds(r0, _CH), :] = (a * dis).astype(jnp.bfloat16)
        y = jnp.dot(x_ref[pl.ds(r0, _CH), :], w0,
                    preferred_element_type=jnp.float32)
        z = dis * y
        zb_s[pl.ds(r0, _CH), :] = z.astype(jnp.bfloat16)
        disr_s[pl.ds(r0, _CH), :] = jnp.broadcast_to(dis, z.shape)

        @pl.when(k + _NBUF < nc)
        def _():
            _start(k + _NBUF, slot)

        return carry

    jax.lax.fori_loop(0, nc, _p0, 0)

    def _p1(i, carry):
        r0 = pl.multiple_of(i * _BM, _BM)
        ab = ab_s[pl.ds(r0, _BM), :]
        acc = jnp.dot(ab, zb_s[...], preferred_element_type=jnp.float32)
        dis = disr_s[pl.ds(r0, _BM), :]
        zblk = zb_s[pl.ds(r0, _BM), :].astype(jnp.float32)
        emb = jnp.maximum(acc + la * dis * zblk + b0, 0.0)
        h_ref[pl.ds(r0, _BM), :] = emb
        ln_ref[pl.ds(r0, _BM), :] = jnp.dot(
            emb, mlpw, preferred_element_type=jnp.float32) + mlpb
        wdec_p = params_ref[0:128, :]
        w = jnp.dot(emb, wdec_p, preferred_element_type=jnp.float32)
        zw = dis * w
        zwb_s[pl.ds(r0, _BM), :] = zw.astype(jnp.bfloat16)
        return carry

    jax.lax.fori_loop(0, nb, _p1, 0)

    def _p2(i, carry):
        r0 = pl.multiple_of(i * _BM, _BM)
        ab = ab_s[pl.ds(r0, _BM), :]
        acc = jnp.dot(ab, zwb_s[...], preferred_element_type=jnp.float32)
        nc_ = logits_ref.shape[1]
        zw = zwb_s[pl.ds(r0, _BM), 0:nc_].astype(jnp.float32)
        disc = disr_s[pl.ds(r0, _BM), 0:nc_]
        logits_ref[pl.ds(r0, _BM), :] = acc[:, 0:nc_] + la * disc * zw + bdec
        return carry

    jax.lax.fori_loop(0, nb, _p2, 0)


def kernel(x, adj, loop_att, W0, b0, W_dec, b_dec, mlp_W, mlp_b):
    N, D = x.shape[1], x.shape[2]
    C = W_dec.shape[1]

    A = adj.reshape(N, N)
    x2 = x.reshape(N, D)

    P = jnp.zeros((392, 128), jnp.float32)
    P = P.at[0:128, 0:C].set(W_dec)
    P = P.at[128:256, 0:C].set(mlp_W)
    P = P.at[256:384, :].set(W0)
    P = P.at[384, :].set(b0)
    P = P.at[385, 0:C].set(b_dec)
    P = P.at[386, 0:C].set(mlp_b)
    P = P.at[387, 0].set(loop_att)

    h, ln, logits = pl.pallas_call(
        _body,
        in_specs=[
            pl.BlockSpec(memory_space=pltpu.MemorySpace.VMEM),
            pl.BlockSpec(memory_space=pltpu.MemorySpace.VMEM),
            pl.BlockSpec(memory_space=pltpu.MemorySpace.HBM),
        ],
        out_specs=[
            pl.BlockSpec(memory_space=pltpu.MemorySpace.VMEM),
            pl.BlockSpec(memory_space=pltpu.MemorySpace.VMEM),
            pl.BlockSpec(memory_space=pltpu.MemorySpace.VMEM),
        ],
        out_shape=[
            jax.ShapeDtypeStruct((N, D), jnp.float32),
            jax.ShapeDtypeStruct((N, C), jnp.float32),
            jax.ShapeDtypeStruct((N, C), jnp.float32),
        ],
        scratch_shapes=[
            pltpu.VMEM((N, N), jnp.bfloat16),
            pltpu.VMEM((N, D), jnp.bfloat16),
            pltpu.VMEM((N, D), jnp.float32),
            pltpu.VMEM((N, 128), jnp.bfloat16),
            pltpu.VMEM((_NBUF, _CH, N), jnp.float32),
            pltpu.SemaphoreType.DMA((_NBUF,)),
        ],
    )(P, x2, A)

    return (logits, ln[None], h)
